# baseline (speedup 1.0000x reference)
import jax
import jax.numpy as jnp
from jax.experimental import pallas as pl

_TOK_BLOCK = 2048

_INV_SQRT2 = 0.7071067811865476


def _gelu_exact(x):
    return 0.5 * x * (1.0 + jax.lax.erf(x * _INV_SQRT2))


def _dot_nk(a, b):
    return jax.lax.dot_general(
        a.astype(jnp.bfloat16), b.astype(jnp.bfloat16),
        dimension_numbers=(((1,), (1,)), ((), ())),
        preferred_element_type=jnp.float32,
    )


def _fused_kernel(x_ref, pm_ref, wl_ref, w1_ref, w2_ref, w3_ref,
                  out_ref, mask_ref, curr_ref):
    x0 = x_ref[0]
    h = _gelu_exact(_dot_nk(x0, wl_ref[...]))
    h = _gelu_exact(_dot_nk(h, w1_ref[...]))
    h = _gelu_exact(_dot_nk(h, w2_ref[...]))
    logit = _dot_nk(w3_ref[...], h)
    curr = jax.nn.sigmoid(logit) * pm_ref[0, :, :]
    keep = (curr > 0.5).astype(jnp.float32)
    curr_m = keep + 1e-10
    curr_ref[0, :, :] = curr_m
    mask_ref[0, :, :] = curr_m.astype(jnp.int32)
    out_ref[0] = x0 * jnp.transpose(curr_m)


def kernel(input_feature, attention_mask, prev_m, W_L, W1, W2, W3):
    B, S, D = input_feature.shape
    T = _TOK_BLOCK
    SB = S // T
    grid = (B * SB,)

    pm = prev_m.reshape(B * SB, 1, T)

    out, mask, curr_m = pl.pallas_call(
        _fused_kernel,
        grid=grid,
        in_specs=[
            pl.BlockSpec((1, T, D), lambda i: (i // SB, i % SB, 0)),
            pl.BlockSpec((1, 1, T), lambda i: (i, 0, 0)),
            pl.BlockSpec(W_L.shape, lambda i: (0, 0)),
            pl.BlockSpec(W1.shape, lambda i: (0, 0)),
            pl.BlockSpec(W2.shape, lambda i: (0, 0)),
            pl.BlockSpec(W3.shape, lambda i: (0, 0)),
        ],
        out_specs=[
            pl.BlockSpec((1, T, D), lambda i: (i // SB, i % SB, 0)),
            pl.BlockSpec((1, 1, T), lambda i: (i, 0, 0)),
            pl.BlockSpec((1, 1, T), lambda i: (i, 0, 0)),
        ],
        out_shape=[
            jax.ShapeDtypeStruct((B, S, D), jnp.float32),
            jax.ShapeDtypeStruct((B * SB, 1, T), jnp.int32),
            jax.ShapeDtypeStruct((B * SB, 1, T), jnp.float32),
        ],
    )(input_feature, pm, W_L, W1, W2, W3)

    return (out, mask.reshape(B, S), curr_m.reshape(B, S))


# speedup vs baseline: 1.0425x; 1.0425x over previous
import jax
import jax.numpy as jnp
from jax.experimental import pallas as pl

_TOK_BLOCK = 4096

_INV_SQRT2 = 0.7071067811865476


def _gelu_exact(x):
    return 0.5 * x * (1.0 + jax.lax.erf(x * _INV_SQRT2))


def _dot_nk(a, b):
    return jax.lax.dot_general(
        a.astype(jnp.bfloat16), b.astype(jnp.bfloat16),
        dimension_numbers=(((1,), (1,)), ((), ())),
        preferred_element_type=jnp.float32,
    )


def _fused_kernel(x_ref, pm_ref, wl_ref, w1_ref, w2_ref, w3_ref,
                  out_ref, mask_ref, curr_ref):
    x0 = x_ref[0]
    h = _gelu_exact(_dot_nk(x0, wl_ref[...]))
    h = _gelu_exact(_dot_nk(h, w1_ref[...]))
    h = _gelu_exact(_dot_nk(h, w2_ref[...]))
    logit = _dot_nk(w3_ref[...], h)
    curr = jax.nn.sigmoid(logit) * pm_ref[0, :, :]
    keep = (curr > 0.5).astype(jnp.float32)
    curr_m = keep + 1e-10
    curr_ref[0, :, :] = curr_m
    mask_ref[0, :, :] = curr_m.astype(jnp.int32)
    out_ref[0] = x0 * jnp.transpose(curr_m)


def kernel(input_feature, attention_mask, prev_m, W_L, W1, W2, W3):
    B, S, D = input_feature.shape
    T = _TOK_BLOCK
    SB = S // T
    grid = (B * SB,)

    pm = prev_m.reshape(B * SB, 1, T)

    out, mask, curr_m = pl.pallas_call(
        _fused_kernel,
        grid=grid,
        in_specs=[
            pl.BlockSpec((1, T, D), lambda i: (i // SB, i % SB, 0)),
            pl.BlockSpec((1, 1, T), lambda i: (i, 0, 0)),
            pl.BlockSpec(W_L.shape, lambda i: (0, 0)),
            pl.BlockSpec(W1.shape, lambda i: (0, 0)),
            pl.BlockSpec(W2.shape, lambda i: (0, 0)),
            pl.BlockSpec(W3.shape, lambda i: (0, 0)),
        ],
        out_specs=[
            pl.BlockSpec((1, T, D), lambda i: (i // SB, i % SB, 0)),
            pl.BlockSpec((1, 1, T), lambda i: (i, 0, 0)),
            pl.BlockSpec((1, 1, T), lambda i: (i, 0, 0)),
        ],
        out_shape=[
            jax.ShapeDtypeStruct((B, S, D), jnp.float32),
            jax.ShapeDtypeStruct((B * SB, 1, T), jnp.int32),
            jax.ShapeDtypeStruct((B * SB, 1, T), jnp.float32),
        ],
    )(input_feature, pm, W_L, W1, W2, W3)

    return (out, mask.reshape(B, S), curr_m.reshape(B, S))


# speedup vs baseline: 1.1663x; 1.1188x over previous
import jax
import jax.numpy as jnp
from jax.experimental import pallas as pl

_S_BLOCK = 1024

_INV_SQRT2 = 0.7071067811865476


def _gelu_exact(x):
    return 0.5 * x * (1.0 + jax.lax.erf(x * _INV_SQRT2))


def _dot_nk(a, b):
    return jax.lax.dot_general(
        a.astype(jnp.bfloat16), b.astype(jnp.bfloat16),
        dimension_numbers=(((1,), (1,)), ((), ())),
        preferred_element_type=jnp.float32,
    )


def _fused_kernel(x_ref, pm_ref, wl_ref, w1_ref, w2_ref, w3_ref,
                  out_ref, mask_ref, curr_ref):
    B, T, D = x_ref.shape
    x0 = x_ref[...].reshape(B * T, D)
    h = _gelu_exact(_dot_nk(x0, wl_ref[...]))
    h = _gelu_exact(_dot_nk(h, w1_ref[...]))
    h = _gelu_exact(_dot_nk(h, w2_ref[...]))
    logit = _dot_nk(w3_ref[...], h).reshape(B, T)
    curr = jax.nn.sigmoid(logit) * pm_ref[...]
    keep = (curr > 0.5).astype(jnp.float32)
    curr_m = keep + 1e-10
    curr_ref[...] = curr_m
    mask_ref[...] = curr_m.astype(jnp.int32)
    out_ref[...] = (x0 * jnp.transpose(curr_m.reshape(1, B * T))).reshape(B, T, D)


def kernel(input_feature, attention_mask, prev_m, W_L, W1, W2, W3):
    B, S, D = input_feature.shape
    T = _S_BLOCK
    grid = (S // T,)

    out, mask, curr_m = pl.pallas_call(
        _fused_kernel,
        grid=grid,
        in_specs=[
            pl.BlockSpec((B, T, D), lambda j: (0, j, 0)),
            pl.BlockSpec((B, T), lambda j: (0, j)),
            pl.BlockSpec(W_L.shape, lambda j: (0, 0)),
            pl.BlockSpec(W1.shape, lambda j: (0, 0)),
            pl.BlockSpec(W2.shape, lambda j: (0, 0)),
            pl.BlockSpec(W3.shape, lambda j: (0, 0)),
        ],
        out_specs=[
            pl.BlockSpec((B, T, D), lambda j: (0, j, 0)),
            pl.BlockSpec((B, T), lambda j: (0, j)),
            pl.BlockSpec((B, T), lambda j: (0, j)),
        ],
        out_shape=[
            jax.ShapeDtypeStruct((B, S, D), jnp.float32),
            jax.ShapeDtypeStruct((B, S), jnp.int32),
            jax.ShapeDtypeStruct((B, S), jnp.float32),
        ],
    )(input_feature, prev_m, W_L, W1, W2, W3)

    return (out, mask, curr_m)
